# baseline (speedup 1.0000x reference)
import functools
import jax
import jax.numpy as jnp
from jax import lax
from jax.experimental import pallas as pl
from jax.experimental.pallas import tpu as pltpu
from jax.experimental.pallas import tpu_sc as plsc

N_LABELS_K = 19
HW = 512 * 512
N_WORKERS = 32
PER_W = HW // 4
P = 1024
N_CHUNKS = PER_W // P


def _sc_body(label_hbm, out_hbm, buf0, buf1, lab0, lab1, sem0, sem1):
    cid = lax.axis_index("c")
    sid = lax.axis_index("s")
    wid = sid * 2 + cid
    n = wid // 4
    base = (wid % 4) * PER_W

    zeros16 = jnp.zeros((16,), jnp.float32)
    ones16 = jnp.full((16,), 1.0, jnp.float32)
    iota16 = lax.iota(jnp.int32, 16)

    bufs = (buf0, buf1)
    labs = (lab0, lab1)
    sems = (sem0, sem1)

    def init_row(r, carry):
        for b in range(2):
            for j in range(P // 16):
                bufs[b][r, pl.ds(j * 16, 16)] = zeros16
        return carry

    lax.fori_loop(0, N_LABELS_K, init_row, 0)

    def scatter(b, val16):
        for i in range(P // 16):
            l16 = labs[b][pl.ds(i * 16, 16)]
            p16 = iota16 + (i * 16)
            plsc.store_scatter(bufs[b], [l16, p16], val16)

    for b in range(2):
        pos = base + b * P
        pltpu.sync_copy(label_hbm.at[n, pl.ds(pos, P)], labs[b])
        scatter(b, ones16)
        pltpu.async_copy(bufs[b], out_hbm.at[n, :, pl.ds(pos, P)], sems[b])

    def pair(g2, carry):
        for b in range(2):
            pos = base + (g2 * 2 + b) * P
            pltpu.make_async_copy(
                bufs[b], out_hbm.at[n, :, pl.ds(base, P)], sems[b]
            ).wait()
            scatter(b, zeros16)
            pltpu.sync_copy(label_hbm.at[n, pl.ds(pos, P)], labs[b])
            scatter(b, ones16)
            pltpu.async_copy(bufs[b], out_hbm.at[n, :, pl.ds(pos, P)], sems[b])
        return carry

    lax.fori_loop(1, N_CHUNKS // 2, pair, 0)

    for b in range(2):
        pltpu.make_async_copy(
            bufs[b], out_hbm.at[n, :, pl.ds(base, P)], sems[b]
        ).wait()


def kernel(label):
    N, H, W = label.shape
    label_flat = label.reshape(N, H * W)
    sc_call = functools.partial(
        pl.kernel,
        mesh=plsc.VectorSubcoreMesh(core_axis_name="c", subcore_axis_name="s"),
        compiler_params=pltpu.CompilerParams(
            use_tc_tiling_on_sc=False, needs_layout_passes=False
        ),
        out_type=jax.ShapeDtypeStruct((N, N_LABELS_K, H * W), jnp.float32),
        scratch_types=[
            pltpu.VMEM((N_LABELS_K, P), jnp.float32),
            pltpu.VMEM((N_LABELS_K, P), jnp.float32),
            pltpu.VMEM((P,), jnp.int32),
            pltpu.VMEM((P,), jnp.int32),
            pltpu.SemaphoreType.DMA,
            pltpu.SemaphoreType.DMA,
        ],
    )(_sc_body)
    out = sc_call(label_flat)
    return out.reshape(N, N_LABELS_K, H, W)


# speedup vs baseline: 3.4512x; 3.4512x over previous
import jax
import jax.numpy as jnp
from jax.experimental import pallas as pl

N_LABELS_K = 19


def _onehot_body(label_ref, out_ref):
    c = pl.program_id(1)
    out_ref[0, 0] = (label_ref[0] == c).astype(jnp.float32)


def kernel(label):
    N, H, W = label.shape
    return pl.pallas_call(
        _onehot_body,
        grid=(N, N_LABELS_K),
        in_specs=[pl.BlockSpec((1, H, W), lambda n, c: (n, 0, 0))],
        out_specs=pl.BlockSpec((1, 1, H, W), lambda n, c: (n, c, 0, 0)),
        out_shape=jax.ShapeDtypeStruct((N, N_LABELS_K, H, W), jnp.float32),
    )(label)


# speedup vs baseline: 5.9060x; 1.7113x over previous
import jax
import jax.numpy as jnp
from jax.experimental import pallas as pl

N_LABELS_K = 19
H_BLK = 256


def _onehot_body(label_ref, out_ref):
    lab = label_ref[0]
    cls = jax.lax.broadcasted_iota(jnp.int32, (N_LABELS_K, H_BLK, 512), 0)
    out_ref[0] = (lab[None, :, :] == cls).astype(jnp.float32)


def kernel(label):
    N, H, W = label.shape
    grid = (N, H // H_BLK)
    return pl.pallas_call(
        _onehot_body,
        grid=grid,
        in_specs=[pl.BlockSpec((1, H_BLK, W), lambda n, h: (n, h, 0))],
        out_specs=pl.BlockSpec((1, N_LABELS_K, H_BLK, W), lambda n, h: (n, 0, h, 0)),
        out_shape=jax.ShapeDtypeStruct((N, N_LABELS_K, H, W), jnp.float32),
    )(label)
